# baseline (speedup 1.0000x reference)
import functools

import jax
import jax.numpy as jnp
from jax import lax
from jax.experimental import pallas as pl
from jax.experimental.pallas import tpu as pltpu
from jax.experimental.pallas import tpu_sc as plsc

P = 262144
D = 64
NB = 3
NCLS = 5
NC = 2
NS = 16
L = 16
NW = NC * NS
PW = P // NW
C = 128
NCH = PW // C
G = C // L


def _sc_body(u_hbm, v_hbm, ui_hbm, vi_hbm, w_hbm, ws_hbm, out_hbm,
             ui_v, vi_v, u_rows, v_rows, wtmp_v, ws_v,
             b0_v, b1_v, b2_v, out_v, su, sv, so):
    wid = lax.axis_index("s") * NC + lax.axis_index("c")
    base = wid * PW

    pltpu.sync_copy(ui_hbm.at[pl.ds(base, PW)], ui_v)
    pltpu.sync_copy(vi_hbm.at[pl.ds(base, PW)], vi_v)

    pltpu.sync_copy(w_hbm, wtmp_v.at[pl.ds(0, NB * D)])
    pltpu.sync_copy(ws_hbm, wtmp_v.at[pl.ds(NB * D, L)])
    vec = wtmp_v[pl.ds(NB * D, L)]
    for j in range(NB * NCLS):
        ws_v[j] = vec[j]
    wv = [
        [wtmp_v[pl.ds(i * D + k * L, L)] for k in range(D // L)]
        for i in range(NB)
    ]

    lane = lax.iota(jnp.int32, L)
    zero = jnp.zeros((L,), jnp.int32)
    m15 = lane == (L - 1)
    colbufs = (b0_v, b1_v, b2_v)

    def start_in(g, b):
        pltpu.async_copy(
            u_hbm.at[ui_v.at[pl.ds(g * C, C)]], u_rows.at[b], su.at[b]
        )
        pltpu.async_copy(
            v_hbm.at[vi_v.at[pl.ds(g * C, C)]], v_rows.at[b], sv.at[b]
        )

    def wait_in(b):
        pltpu.make_async_copy(
            u_hbm.at[ui_v.at[pl.ds(0, C)]], u_rows.at[b], su.at[b]
        ).wait()
        pltpu.make_async_copy(
            v_hbm.at[vi_v.at[pl.ds(0, C)]], v_rows.at[b], sv.at[b]
        ).wait()

    def wait_out(b):
        pltpu.make_async_copy(
            out_v.at[b], out_hbm.at[:, pl.ds(base, C)], so.at[b]
        ).wait()

    def compute(gb, b):
        @plsc.parallel_loop(0, C, unroll=10)
        def pair_body(p):
            e = [
                u_rows[b, p, pl.ds(k * L, L)] * v_rows[b, p, pl.ds(k * L, L)]
                for k in range(D // L)
            ]
            pidx = zero + p
            for i in range(NB):
                t = e[0] * wv[i][0]
                for k in range(1, D // L):
                    t = t + e[k] * wv[i][k]
                s = plsc.cumsum(t)
                plsc.store_scatter(colbufs[i], [pidx], s, mask=m15)

        def grp_body(gr):
            acc = [colbufs[i][pl.ds(gr * L, L)] for i in range(NB)]
            logits = []
            for c in range(NCLS):
                lg = acc[0] * ws_v[c]
                for i in range(1, NB):
                    lg = lg + acc[i] * ws_v[i * NCLS + c]
                logits.append(lg)
            m = logits[0]
            for c in range(1, NCLS):
                m = jnp.maximum(m, logits[c])
            ex = [jnp.exp(lg - m) for lg in logits]
            s = ex[0]
            for c in range(1, NCLS):
                s = s + ex[c]
            inv = 1.0 / s
            for c in range(NCLS):
                out_v[b, c, pl.ds(gr * L, L)] = ex[c] * inv

        for gr in range(G):
            grp_body(gr)

    start_in(0, 0)

    def chunk2_body(g2, _):
        for b in (0, 1):
            gb = g2 * 2 + b

            @pl.when(gb + 1 < NCH)
            def _():
                start_in(gb + 1, 1 - b)

            wait_in(b)

            @pl.when(gb >= 2)
            def _():
                wait_out(b)

            compute(gb, b)
            pltpu.async_copy(
                out_v.at[b], out_hbm.at[:, pl.ds(base + gb * C, C)], so.at[b]
            )
        return _

    lax.fori_loop(0, NCH // 2, chunk2_body, None)
    wait_out(0)
    wait_out(1)


def kernel(u_features, v_features, u_indices, v_indices, weights, weights_scalars):
    w_flat = weights.reshape(-1)
    ws_flat = jnp.concatenate(
        [weights_scalars.reshape(-1), jnp.zeros((1,), jnp.float32)]
    )
    mesh = plsc.VectorSubcoreMesh(core_axis_name="c", subcore_axis_name="s")
    f = functools.partial(
        pl.kernel,
        mesh=mesh,
        out_type=jax.ShapeDtypeStruct((NCLS, P), jnp.float32),
        compiler_params=pltpu.CompilerParams(
            use_tc_tiling_on_sc=False, needs_layout_passes=False
        ),
        scratch_types=[
            pltpu.VMEM((PW,), jnp.int32),
            pltpu.VMEM((PW,), jnp.int32),
            pltpu.VMEM((2, C, D), jnp.float32),
            pltpu.VMEM((2, C, D), jnp.float32),
            pltpu.VMEM((NB * D + L,), jnp.float32),
            pltpu.SMEM((NB * NCLS,), jnp.float32),
            pltpu.VMEM((C,), jnp.float32),
            pltpu.VMEM((C,), jnp.float32),
            pltpu.VMEM((C,), jnp.float32),
            pltpu.VMEM((2, NCLS, C), jnp.float32),
            pltpu.SemaphoreType.DMA((2,)),
            pltpu.SemaphoreType.DMA((2,)),
            pltpu.SemaphoreType.DMA((2,)),
        ],
    )(_sc_body)
    out5 = f(u_features, v_features, u_indices, v_indices, w_flat, ws_flat)
    return out5.T


# speedup vs baseline: 1.0163x; 1.0163x over previous
import functools

import jax
import jax.numpy as jnp
from jax import lax
from jax.experimental import pallas as pl
from jax.experimental.pallas import tpu as pltpu
from jax.experimental.pallas import tpu_sc as plsc

P = 262144
D = 64
NB = 3
NCLS = 5
NC = 2
NS = 16
L = 16
NW = NC * NS
PW = P // NW
C = 128
NCH = PW // C
G = C // L


def _sc_body(u_hbm, v_hbm, ui_hbm, vi_hbm, w_hbm, ws_hbm, out_hbm,
             ui_v, vi_v, u_rows, v_rows, wtmp_v, ws_v,
             b0_v, b1_v, b2_v, out_v, su, sv, so):
    wid = lax.axis_index("s") * NC + lax.axis_index("c")
    base = wid * PW

    pltpu.sync_copy(ui_hbm.at[pl.ds(base, PW)], ui_v)
    pltpu.sync_copy(vi_hbm.at[pl.ds(base, PW)], vi_v)

    pltpu.sync_copy(w_hbm, wtmp_v.at[pl.ds(0, NB * D)])
    pltpu.sync_copy(ws_hbm, wtmp_v.at[pl.ds(NB * D, L)])
    vec = wtmp_v[pl.ds(NB * D, L)]
    for j in range(NB * NCLS):
        ws_v[j] = vec[j]
    wv = [
        [wtmp_v[pl.ds(i * D + k * L, L)] for k in range(D // L)]
        for i in range(NB)
    ]

    lane = lax.iota(jnp.int32, L)
    zero = jnp.zeros((L,), jnp.int32)
    m15 = lane == (L - 1)
    colbufs = (b0_v, b1_v, b2_v)

    def start_in(g, b):
        pltpu.async_copy(
            u_hbm.at[ui_v.at[pl.ds(g * C, C)]], u_rows.at[b], su.at[b]
        )
        pltpu.async_copy(
            v_hbm.at[vi_v.at[pl.ds(g * C, C)]], v_rows.at[b], sv.at[b]
        )

    def wait_in(b):
        pltpu.make_async_copy(
            u_hbm.at[ui_v.at[pl.ds(0, C)]], u_rows.at[b], su.at[b]
        ).wait()
        pltpu.make_async_copy(
            v_hbm.at[vi_v.at[pl.ds(0, C)]], v_rows.at[b], sv.at[b]
        ).wait()

    def wait_out(b):
        pltpu.make_async_copy(
            out_v.at[b], out_hbm.at[:, pl.ds(base, C)], so.at[b]
        ).wait()

    def compute(gb, b):
        @plsc.parallel_loop(0, C, unroll=8)
        def pair_body(p):
            e = [
                u_rows[b, p, pl.ds(k * L, L)] * v_rows[b, p, pl.ds(k * L, L)]
                for k in range(D // L)
            ]
            pidx = zero + p
            for i in range(NB):
                t = e[0] * wv[i][0]
                for k in range(1, D // L):
                    t = t + e[k] * wv[i][k]
                s = plsc.cumsum(t)
                plsc.store_scatter(colbufs[i], [pidx], s, mask=m15)

        def grp_body(gr):
            acc = [colbufs[i][pl.ds(gr * L, L)] for i in range(NB)]
            logits = []
            for c in range(NCLS):
                lg = acc[0] * ws_v[c]
                for i in range(1, NB):
                    lg = lg + acc[i] * ws_v[i * NCLS + c]
                logits.append(lg)
            m = logits[0]
            for c in range(1, NCLS):
                m = jnp.maximum(m, logits[c])
            ex = [jnp.exp(lg - m) for lg in logits]
            s = ex[0]
            for c in range(1, NCLS):
                s = s + ex[c]
            inv = 1.0 / s
            for c in range(NCLS):
                out_v[b, c, pl.ds(gr * L, L)] = ex[c] * inv

        for gr in range(G):
            grp_body(gr)

    start_in(0, 0)

    def chunk2_body(g2, _):
        for b in (0, 1):
            gb = g2 * 2 + b

            @pl.when(gb + 1 < NCH)
            def _():
                start_in(gb + 1, 1 - b)

            wait_in(b)

            @pl.when(gb >= 2)
            def _():
                wait_out(b)

            compute(gb, b)
            pltpu.async_copy(
                out_v.at[b], out_hbm.at[:, pl.ds(base + gb * C, C)], so.at[b]
            )
        return _

    lax.fori_loop(0, NCH // 2, chunk2_body, None)
    wait_out(0)
    wait_out(1)


def kernel(u_features, v_features, u_indices, v_indices, weights, weights_scalars):
    w_flat = weights.reshape(-1)
    ws_flat = jnp.concatenate(
        [weights_scalars.reshape(-1), jnp.zeros((1,), jnp.float32)]
    )
    mesh = plsc.VectorSubcoreMesh(core_axis_name="c", subcore_axis_name="s")
    f = functools.partial(
        pl.kernel,
        mesh=mesh,
        out_type=jax.ShapeDtypeStruct((NCLS, P), jnp.float32),
        compiler_params=pltpu.CompilerParams(
            use_tc_tiling_on_sc=False, needs_layout_passes=False
        ),
        scratch_types=[
            pltpu.VMEM((PW,), jnp.int32),
            pltpu.VMEM((PW,), jnp.int32),
            pltpu.VMEM((2, C, D), jnp.float32),
            pltpu.VMEM((2, C, D), jnp.float32),
            pltpu.VMEM((NB * D + L,), jnp.float32),
            pltpu.SMEM((NB * NCLS,), jnp.float32),
            pltpu.VMEM((C,), jnp.float32),
            pltpu.VMEM((C,), jnp.float32),
            pltpu.VMEM((C,), jnp.float32),
            pltpu.VMEM((2, NCLS, C), jnp.float32),
            pltpu.SemaphoreType.DMA((2,)),
            pltpu.SemaphoreType.DMA((2,)),
            pltpu.SemaphoreType.DMA((2,)),
        ],
    )(_sc_body)
    out5 = f(u_features, v_features, u_indices, v_indices, w_flat, ws_flat)
    return out5.T
